# baseline (speedup 1.0000x reference)
import functools

import jax
import jax.numpy as jnp
from jax import lax
from jax.experimental import pallas as pl
from jax.experimental.pallas import tpu as pltpu
from jax.experimental.pallas import tpu_sc as plsc

B = 16384
D = 64
NC = 2
NS = 16
NW = NC * NS
BPW = B // NW
CH = 128
NCH = BPW // CH

_mesh = plsc.VectorSubcoreMesh(core_axis_name="c", subcore_axis_name="s")


@functools.partial(
    pl.kernel,
    mesh=_mesh,
    compiler_params=pltpu.CompilerParams(use_tc_tiling_on_sc=False),
    out_type=(
        jax.ShapeDtypeStruct((NW, BPW, D), jnp.float32),
        jax.ShapeDtypeStruct((NW, BPW, D), jnp.float32),
        jax.ShapeDtypeStruct((NW, BPW, D), jnp.float32),
    ),
    scratch_types=(
        pltpu.VMEM((NCH, CH), jnp.int32),
        pltpu.VMEM((NCH, CH), jnp.int32),
        pltpu.VMEM((NCH, CH), jnp.int32),
        pltpu.VMEM((BPW, D), jnp.float32),
        pltpu.VMEM((BPW, D), jnp.float32),
        pltpu.VMEM((BPW, D), jnp.float32),
        pltpu.SemaphoreType.DMA,
        pltpu.SemaphoreType.DMA,
        pltpu.SemaphoreType.DMA,
        pltpu.SemaphoreType.DMA,
    ),
)
def _gather3(uid, iid, cid, ut, it, ct, ou, oi, oc,
             uidx, iidx, cidx, urows, irows, crows, s0, s1, s2, s3):
    wid = lax.axis_index("s") * NC + lax.axis_index("c")
    pltpu.sync_copy(uid.at[wid], uidx)
    pltpu.sync_copy(iid.at[wid], iidx)
    pltpu.sync_copy(cid.at[wid], cidx)
    tables = (
        (uidx, urows, ut, s0, ou),
        (iidx, irows, it, s1, oi),
        (cidx, crows, ct, s2, oc),
    )
    pending = []
    for idx_v, rows_v, tab, sem, _ in tables:
        cps = []
        for j in range(NCH):
            cps.append(
                pltpu.async_copy(
                    tab.at[idx_v.at[j]],
                    rows_v.at[pl.ds(j * CH, CH)],
                    sem,
                )
            )
        pending.append(cps)
    out_cps = []
    for (idx_v, rows_v, tab, sem, out), cps in zip(tables, pending):
        for cp in cps:
            cp.wait()
        out_cps.append(pltpu.async_copy(rows_v, out.at[wid], s3))
    for cp in out_cps:
        cp.wait()


def kernel(user_id, item_id, category_id, user_table, item_table, cat_table):
    uid = user_id.reshape(NW, NCH, CH)
    iid = item_id.reshape(NW, NCH, CH)
    cid = category_id.reshape(NW, NCH, CH)
    ou, oi, oc = _gather3(uid, iid, cid, user_table, item_table, cat_table)
    return (ou.reshape(B, D), oi.reshape(B, D), oc.reshape(B, D))
